# baseline (speedup 1.0000x reference)
import jax
import jax.numpy as jnp
from jax.experimental import pallas as pl

N = 131072
D_IN = 512
D_OUT = 32
BLK = 2048


def _body(x_ref, wt_ref, b_ref, o_ref):
    acc = jax.lax.dot_general(
        x_ref[:], wt_ref[:],
        (((1,), (0,)), ((), ())),
        preferred_element_type=jnp.float32,
    )
    o_ref[:] = jnp.maximum(acc + b_ref[:], 0.0)


def kernel(x, W1, b1):
    wt = W1.T
    grid = (N // BLK,)
    return pl.pallas_call(
        _body,
        grid=grid,
        in_specs=[
            pl.BlockSpec((BLK, D_IN), lambda i: (i, 0)),
            pl.BlockSpec((D_IN, D_OUT), lambda i: (0, 0)),
            pl.BlockSpec((D_OUT,), lambda i: (0,)),
        ],
        out_specs=pl.BlockSpec((BLK, D_OUT), lambda i: (i, 0)),
        out_shape=jax.ShapeDtypeStruct((N, D_OUT), jnp.float32),
    )(x, wt, b1)


# speedup vs baseline: 1.1087x; 1.1087x over previous
import jax
import jax.numpy as jnp
from jax.experimental import pallas as pl

N = 131072
D_IN = 512
D_OUT = 32
BLK = 8192


def _body(x_ref, wt_ref, b_ref, o_ref):
    acc = jax.lax.dot_general(
        x_ref[:], wt_ref[:],
        (((1,), (0,)), ((), ())),
        preferred_element_type=jnp.float32,
    )
    o_ref[:] = jnp.maximum(acc + b_ref[:], 0.0)


def kernel(x, W1, b1):
    wt = W1.T
    grid = (N // BLK,)
    return pl.pallas_call(
        _body,
        grid=grid,
        in_specs=[
            pl.BlockSpec((BLK, D_IN), lambda i: (i, 0)),
            pl.BlockSpec((D_IN, D_OUT), lambda i: (0, 0)),
            pl.BlockSpec((D_OUT,), lambda i: (0,)),
        ],
        out_specs=pl.BlockSpec((BLK, D_OUT), lambda i: (i, 0)),
        out_shape=jax.ShapeDtypeStruct((N, D_OUT), jnp.float32),
    )(x, wt, b1)
